# Initial kernel scaffold; baseline (speedup 1.0000x reference)
import jax
import jax.numpy as jnp
from jax.experimental import pallas as pl


def kernel(x, edge_index, Wl1, bl1, Wr1, br1, att1, bias1, g1, b1, m1, v1, Wl2, bl2, Wr2, br2, att2, bias2, g2, b2, m2, v2, Wc, bc):
    raise NotImplementedError("write your pallas kernel here")


# speedup vs baseline: 25.9918x; 25.9918x over previous
import functools

import jax
import jax.numpy as jnp
from jax import lax
from jax.experimental import pallas as pl
from jax.experimental.pallas import tpu as pltpu
from jax.experimental.pallas import tpu_sc as plsc

_NODES = 10000
_E = 320000
_EN = _E + _NODES
_NW = 32
_BLK = 128
_NB = 81
_EP = _NW * _NB * _BLK
_TSLICE = 624
_TSPAN = 640
_EPS_BN = 1e-5


def _mm_dual(x, Wl, bl, Wr, br):
    n, k = x.shape
    fo = Wl.shape[1]
    br_blk = 400

    def body(x_ref, wl_ref, bl_ref, wr_ref, br_ref, ol_ref, or_ref):
        xb = x_ref[...]
        ol_ref[...] = jnp.dot(xb, wl_ref[...], preferred_element_type=jnp.float32) + bl_ref[...]
        or_ref[...] = jnp.dot(xb, wr_ref[...], preferred_element_type=jnp.float32) + br_ref[...]

    return pl.pallas_call(
        body,
        grid=(n // br_blk,),
        in_specs=[
            pl.BlockSpec((br_blk, k), lambda i: (i, 0)),
            pl.BlockSpec((k, fo), lambda i: (0, 0)),
            pl.BlockSpec((1, fo), lambda i: (0, 0)),
            pl.BlockSpec((k, fo), lambda i: (0, 0)),
            pl.BlockSpec((1, fo), lambda i: (0, 0)),
        ],
        out_specs=[pl.BlockSpec((br_blk, fo), lambda i: (i, 0))] * 2,
        out_shape=[jax.ShapeDtypeStruct((n, fo), jnp.float32)] * 2,
    )(x, Wl, bl.reshape(1, -1), Wr, br.reshape(1, -1))


def _mm_dual_fused(oa, ob, bias, scale, shift, Wl, bl, Wr, br):
    n, k = oa.shape
    fo = Wl.shape[1]
    br_blk = 400

    def body(oa_ref, ob_ref, bias_ref, sc_ref, sh_ref, wl_ref, bl_ref,
             wr_ref, br_ref, ol_ref, or_ref):
        t = oa_ref[...] + ob_ref[...] + bias_ref[...]
        t = t * sc_ref[...] + sh_ref[...]
        h = jnp.where(t > 0, t, jnp.exp(jnp.minimum(t, 0.0)) - 1.0)
        ol_ref[...] = jnp.dot(h, wl_ref[...], preferred_element_type=jnp.float32) + bl_ref[...]
        or_ref[...] = jnp.dot(h, wr_ref[...], preferred_element_type=jnp.float32) + br_ref[...]

    row = lambda a: a.reshape(1, -1)
    return pl.pallas_call(
        body,
        grid=(n // br_blk,),
        in_specs=[
            pl.BlockSpec((br_blk, k), lambda i: (i, 0)),
            pl.BlockSpec((br_blk, k), lambda i: (i, 0)),
            pl.BlockSpec((1, k), lambda i: (0, 0)),
            pl.BlockSpec((1, k), lambda i: (0, 0)),
            pl.BlockSpec((1, k), lambda i: (0, 0)),
            pl.BlockSpec((k, fo), lambda i: (0, 0)),
            pl.BlockSpec((1, fo), lambda i: (0, 0)),
            pl.BlockSpec((k, fo), lambda i: (0, 0)),
            pl.BlockSpec((1, fo), lambda i: (0, 0)),
        ],
        out_specs=[pl.BlockSpec((br_blk, fo), lambda i: (i, 0))] * 2,
        out_shape=[jax.ShapeDtypeStruct((n, fo), jnp.float32)] * 2,
    )(oa, ob, row(bias), row(scale), row(shift), Wl, bl.reshape(1, -1), Wr,
      br.reshape(1, -1))


def _final_head(oa, ob, bias, scale, shift, wc_row, bc):
    n, k = oa.shape
    br_blk = 400

    def body(oa_ref, ob_ref, bias_ref, sc_ref, sh_ref, wc_ref, bc_ref, y_ref):
        t = oa_ref[...] + ob_ref[...] + bias_ref[...]
        t = t * sc_ref[...] + sh_ref[...]
        h = jnp.where(t > 0, t, jnp.exp(jnp.minimum(t, 0.0)) - 1.0)
        y = jnp.sum(h * wc_ref[...], axis=1, keepdims=True) + bc_ref[...]
        y_ref[...] = jax.nn.sigmoid(y)

    row = lambda a: a.reshape(1, -1)
    return pl.pallas_call(
        body,
        grid=(n // br_blk,),
        in_specs=[
            pl.BlockSpec((br_blk, k), lambda i: (i, 0)),
            pl.BlockSpec((br_blk, k), lambda i: (i, 0)),
            pl.BlockSpec((1, k), lambda i: (0, 0)),
            pl.BlockSpec((1, k), lambda i: (0, 0)),
            pl.BlockSpec((1, k), lambda i: (0, 0)),
            pl.BlockSpec((1, k), lambda i: (0, 0)),
            pl.BlockSpec((1, 1), lambda i: (0, 0)),
        ],
        out_specs=pl.BlockSpec((br_blk, 1), lambda i: (i, 0)),
        out_shape=jax.ShapeDtypeStruct((n, 1), jnp.float32),
    )(oa, ob, row(bias), row(scale), row(shift), wc_row, bc.reshape(1, 1))


def _make_alpha_kernel(H, F, FP):
    C = F // H
    mesh = plsc.VectorSubcoreMesh(core_axis_name="c", subcore_axis_name="s")

    NCH = F // 16

    def body(xl_hbm, xr_hbm, src_hbm, dst_hbm, att_hbm, ea_hbm, den_hbm,
             idx_s, idx_d, xlb, xrb, attv, zbuf, sb, eab, s_sp, den_sp,
             sem0, sem1, sem2):
        cid = lax.axis_index("c")
        sid = lax.axis_index("s")
        wid = sid * 2 + cid
        nstart = sid * _TSLICE
        z16 = jnp.zeros((16,), jnp.float32)
        lane = lax.iota(jnp.int32, 16)

        pltpu.sync_copy(att_hbm, attv)
        att_ch = [attv[pl.ds(k * 16, 16)] for k in range(NCH)]

        def zb(j, carry):
            zbuf[pl.ds(j * 16, 16)] = z16
            return carry
        lax.fori_loop(0, _TSPAN // 16, zb, 0)
        for h in range(H):
            pltpu.sync_copy(zbuf, den_sp[h].at[pl.ds(nstart, _TSPAN)])

        def alpha_group(g):
            vecs = [z16] * H
            for j in range(16):
                e = g * 16 + j
                accs = [z16] * H
                for k in range(NCH):
                    ch = xlb[e, pl.ds(k * 16, 16)] + xrb[e, pl.ds(k * 16, 16)]
                    z = jnp.maximum(ch, 0.2 * ch) * att_ch[k]
                    hh = (k * 16) // C
                    accs[hh] = accs[hh] + z
                for h in range(H):
                    a = accs[h]
                    for _ in range(4):
                        a = a + lax.rev(a, (0,))
                    vecs[h] = jnp.where(lane == j, a, vecs[h])
            return vecs

        def p1_block(b, carry):
            e0 = _E + nstart + b * _BLK
            pltpu.sync_copy(dst_hbm.at[pl.ds(e0, _BLK)], idx_d)
            cp1 = pltpu.async_copy(xl_hbm.at[idx_d], xlb, sem0)
            cp2 = pltpu.async_copy(xr_hbm.at[idx_d], xrb, sem1)
            cp1.wait()
            cp2.wait()

            def g_body(g, c2):
                vecs = alpha_group(g)
                for h in range(H):
                    eab[h][pl.ds(g * 16, 16)] = vecs[h]
                return c2
            lax.fori_loop(0, 8, g_body, 0)
            for h in range(H):
                pltpu.sync_copy(eab[h], s_sp[h].at[idx_d])
            return carry
        lax.fori_loop(0, _TSPAN // _BLK, p1_block, 0)
        plsc.subcore_barrier()

        def p2_block(bb, carry):
            e0 = wid * (_NB * _BLK) + bb * _BLK
            pltpu.sync_copy(src_hbm.at[pl.ds(e0, _BLK)], idx_s)
            pltpu.sync_copy(dst_hbm.at[pl.ds(e0, _BLK)], idx_d)
            cpl = pltpu.async_copy(xl_hbm.at[idx_s], xlb, sem0)
            cpr = pltpu.async_copy(xr_hbm.at[idx_d], xrb, sem1)
            cps = [pltpu.async_copy(s_sp[h].at[idx_d], sb[h], sem2)
                   for h in range(H)]
            cpl.wait()
            cpr.wait()
            for cp in cps:
                cp.wait()

            def g_body(g, c2):
                vecs = alpha_group(g)
                eid = lane + g * 16 + e0
                for h in range(H):
                    sv = sb[h][pl.ds(g * 16, 16)]
                    eav = jnp.where(eid < _EN, jnp.exp(vecs[h] - sv), 0.0)
                    eab[h][pl.ds(g * 16, 16)] = eav
                return c2
            lax.fori_loop(0, 8, g_body, 0)
            for h in range(H):
                pltpu.sync_copy(eab[h], ea_hbm.at[pl.ds(h * _EP + e0, _BLK)])
                pltpu.sync_copy(eab[h], den_sp[h].at[idx_d], add=True)
            return carry
        lax.fori_loop(0, _NB, p2_block, 0)
        plsc.subcore_barrier()

        for h in range(H):
            off = (cid * H + h) * _NODES + nstart
            pltpu.sync_copy(den_sp[h].at[pl.ds(nstart, _TSPAN)], zbuf)
            pltpu.sync_copy(zbuf, den_hbm.at[pl.ds(off, _TSPAN)])

    return pl.kernel(
        body,
        out_type=[
            jax.ShapeDtypeStruct((H * _EP,), jnp.float32),
            jax.ShapeDtypeStruct((2 * H * _NODES,), jnp.float32),
        ],
        mesh=mesh,
        scratch_types=[
            pltpu.VMEM((_BLK,), jnp.int32),
            pltpu.VMEM((_BLK,), jnp.int32),
            pltpu.VMEM((_BLK, FP), jnp.float32),
            pltpu.VMEM((_BLK, FP), jnp.float32),
            pltpu.VMEM((F,), jnp.float32),
            pltpu.VMEM((_TSPAN,), jnp.float32),
            [pltpu.VMEM((_BLK,), jnp.float32) for _ in range(H)],
            [pltpu.VMEM((_BLK,), jnp.float32) for _ in range(H)],
            [pltpu.VMEM_SHARED((_NODES,), jnp.float32) for _ in range(H)],
            [pltpu.VMEM_SHARED((_NODES,), jnp.float32) for _ in range(H)],
            pltpu.SemaphoreType.DMA,
            pltpu.SemaphoreType.DMA,
            pltpu.SemaphoreType.DMA,
        ],
    )


def _make_agg_kernel(H, F, FP):
    C = F // H
    NCH = F // 16
    NCHP = FP // 16
    mesh = plsc.VectorSubcoreMesh(core_axis_name="c", subcore_axis_name="s")

    def body(xl_hbm, src_hbm, dst_hbm, ea_hbm, den_hbm, out_hbm,
             idx_s, idx_d, xlb, msgb, db, eb, d0, d1, dsum, out_sp, den_sp,
             sem0, sem1):
        cid = lax.axis_index("c")
        sid = lax.axis_index("s")
        wid = sid * 2 + cid
        nstart = sid * _TSLICE
        lane = lax.iota(jnp.int32, 16)
        z16 = jnp.zeros((16,), jnp.float32)

        for h in range(H):
            pltpu.sync_copy(den_hbm.at[pl.ds(h * _NODES + nstart, _TSPAN)], d0)
            pltpu.sync_copy(den_hbm.at[pl.ds((H + h) * _NODES + nstart, _TSPAN)], d1)

            def ab(j, carry):
                dsum[pl.ds(j * 16, 16)] = d0[pl.ds(j * 16, 16)] + d1[pl.ds(j * 16, 16)]
                return carry
            lax.fori_loop(0, _TSPAN // 16, ab, 0)
            pltpu.sync_copy(dsum, den_sp[h].at[pl.ds(nstart, _TSPAN)])

        def zrow(e, carry):
            for k in range(NCHP):
                msgb[e, pl.ds(k * 16, 16)] = z16
            return carry
        lax.fori_loop(0, _BLK, zrow, 0)
        for i in range(_TSPAN // _BLK):
            pltpu.sync_copy(msgb, out_sp.at[pl.ds(nstart + i * _BLK, _BLK)])
        plsc.subcore_barrier()

        def blk(bb, carry):
            e0 = wid * (_NB * _BLK) + bb * _BLK
            pltpu.sync_copy(src_hbm.at[pl.ds(e0, _BLK)], idx_s)
            pltpu.sync_copy(dst_hbm.at[pl.ds(e0, _BLK)], idx_d)
            cpx = pltpu.async_copy(xl_hbm.at[idx_s], xlb, sem0)
            cpd = [pltpu.async_copy(den_sp[h].at[idx_d], db[h], sem1)
                   for h in range(H)]
            for h in range(H):
                pltpu.sync_copy(ea_hbm.at[pl.ds(h * _EP + e0, _BLK)], eb[h])
            cpx.wait()
            for cp in cpd:
                cp.wait()

            def g_body(g, c2):
                coefs = []
                for h in range(H):
                    dv = db[h][pl.ds(g * 16, 16)]
                    ev = eb[h][pl.ds(g * 16, 16)]
                    coefs.append(ev / (dv + 1e-16))
                for j in range(16):
                    e = g * 16 + j
                    splats = []
                    for h in range(H):
                        w = jnp.where(lane == j, coefs[h], 0.0)
                        for _ in range(4):
                            w = w + lax.rev(w, (0,))
                        splats.append(w)
                    for k in range(NCH):
                        hh = (k * 16) // C
                        msgb[e, pl.ds(k * 16, 16)] = (
                            xlb[e, pl.ds(k * 16, 16)] * splats[hh])
                return c2
            lax.fori_loop(0, 8, g_body, 0)
            pltpu.sync_copy(msgb, out_sp.at[idx_d], add=True)
            return carry
        lax.fori_loop(0, _NB, blk, 0)
        plsc.subcore_barrier()

        for i in range(_TSPAN // _BLK):
            pltpu.sync_copy(out_sp.at[pl.ds(nstart + i * _BLK, _BLK)], msgb)
            pltpu.sync_copy(msgb, out_hbm.at[cid, pl.ds(nstart + i * _BLK, _BLK)])

    return pl.kernel(
        body,
        out_type=[jax.ShapeDtypeStruct((2, _NODES, FP), jnp.float32)],
        mesh=mesh,
        scratch_types=[
            pltpu.VMEM((_BLK,), jnp.int32),
            pltpu.VMEM((_BLK,), jnp.int32),
            pltpu.VMEM((_BLK, FP), jnp.float32),
            pltpu.VMEM((_BLK, FP), jnp.float32),
            [pltpu.VMEM((_BLK,), jnp.float32) for _ in range(H)],
            [pltpu.VMEM((_BLK,), jnp.float32) for _ in range(H)],
            pltpu.VMEM((_TSPAN,), jnp.float32),
            pltpu.VMEM((_TSPAN,), jnp.float32),
            pltpu.VMEM((_TSPAN,), jnp.float32),
            pltpu.VMEM_SHARED((_NODES, FP), jnp.float32),
            [pltpu.VMEM_SHARED((_NODES,), jnp.float32) for _ in range(H)],
            pltpu.SemaphoreType.DMA,
            pltpu.SemaphoreType.DMA,
        ],
    )


_alpha1 = _make_alpha_kernel(2, 128, 128)
_agg1 = _make_agg_kernel(2, 128, 128)
_alpha2 = _make_alpha_kernel(1, 64, 128)
_agg2 = _make_agg_kernel(1, 64, 128)


def kernel(x, edge_index, Wl1, bl1, Wr1, br1, att1, bias1, g1, b1, m1, v1,
           Wl2, bl2, Wr2, br2, att2, bias2, g2, b2, m2, v2, Wc, bc):
    loops = jnp.arange(_NODES, dtype=edge_index.dtype)
    pad = jnp.zeros((_EP - _EN,), edge_index.dtype)
    src = jnp.concatenate([edge_index[0], loops, pad])
    dst = jnp.concatenate([edge_index[1], loops, pad])

    scale1 = g1 / jnp.sqrt(v1 + _EPS_BN)
    shift1 = b1 - m1 * scale1
    scale2 = g2 / jnp.sqrt(v2 + _EPS_BN)
    shift2 = b2 - m2 * scale2

    xl1, xr1 = _mm_dual(x, Wl1, bl1, Wr1, br1)
    ea1, den1 = _alpha1(xl1, xr1, src, dst, att1.reshape(-1))
    (out1,) = _agg1(xl1, src, dst, ea1, den1)

    zw = jnp.zeros((Wl2.shape[0], 64), jnp.float32)
    zb64 = jnp.zeros((64,), jnp.float32)
    xl2, xr2 = _mm_dual_fused(out1[0], out1[1], bias1, scale1, shift1,
                              jnp.concatenate([Wl2, zw], axis=1),
                              jnp.concatenate([bl2, zb64]),
                              jnp.concatenate([Wr2, zw], axis=1),
                              jnp.concatenate([br2, zb64]))
    ea2, den2 = _alpha2(xl2, xr2, src, dst, att2.reshape(-1))
    (out2,) = _agg2(xl2, src, dst, ea2, den2)

    return _final_head(out2[0], out2[1],
                       jnp.concatenate([bias2, zb64]),
                       jnp.concatenate([scale2, jnp.ones((64,), jnp.float32)]),
                       jnp.concatenate([shift2, zb64]),
                       jnp.concatenate([Wc.reshape(1, -1),
                                        zb64.reshape(1, -1)], axis=1), bc)
